# baseline (speedup 1.0000x reference)
import functools

import jax
import jax.numpy as jnp
from jax import lax
from jax.experimental import pallas as pl
from jax.experimental.pallas import tpu as pltpu
from jax.experimental.pallas import tpu_sc as plsc

VOCAB = 1_000_000
D = 64
DP = 128
B = 4096
T = 200

NC = 2
NS = 16
NW = NC * NS
RPW = B // NW

BLK = 128
NFULL = VOCAB // BLK
TAIL = VOCAB - NFULL * BLK
NBLK = NFULL + 1

CH0, CH1 = 128, 72

L = 16
NV = D // L
SCALE = 1.0 / T


def _relayout_body(src_hbm, tail_hbm, out_hbm, stage_v, outst_v, tail_v,
                   sem_in, sem_out):
  wid = lax.axis_index("s") * NC + lax.axis_index("c")
  lanes = lax.iota(jnp.int32, L)
  NG = D // 8

  def transpose_block(nrows):
    def do_out_row(i, _):
      idx_i = jnp.zeros((L,), jnp.int32) + i
      ih = i // 8
      il = i - ih * 8
      for q in range(NV):
        g = plsc.load_gather(
            stage_v, [(q * L + lanes) // 8, (q * L + lanes) % 8, idx_i])
        outst_v[ih, il, pl.ds(q * L, L)] = g
      return ()
    lax.fori_loop(0, nrows, do_out_row, ())

  @pl.when(wid == 0)
  def _tail():
    pltpu.sync_copy(tail_hbm, tail_v)
    pltpu.sync_copy(tail_v, out_hbm.at[pl.ds(NFULL * BLK, TAIL)])

  nper = NFULL // NW + 1

  def do_block(k, _):
    b = wid + k * NW

    @pl.when(b < NFULL)
    def _full():
      i0 = b * BLK
      for g in range(NG):
        pltpu.async_copy(src_hbm.at[pl.ds(8 * g, 8), pl.ds(i0, BLK)],
                         stage_v.at[g], sem_in)
      for g in range(NG):
        pltpu.make_async_copy(src_hbm.at[pl.ds(8 * g, 8), pl.ds(i0, BLK)],
                              stage_v.at[g], sem_in).wait()
      transpose_block(BLK)
      for j in range(BLK // 8):
        pltpu.async_copy(outst_v.at[j], out_hbm.at[pl.ds(i0 + 8 * j, 8)],
                         sem_out)
      for j in range(BLK // 8):
        pltpu.make_async_copy(outst_v.at[j], out_hbm.at[pl.ds(i0 + 8 * j, 8)],
                              sem_out).wait()

    return ()

  lax.fori_loop(0, nper, do_block, ())


def _gather_body(tok_hbm, table_hbm, out_hbm, idx_v, buf_v, out_v, sem):
  wid = lax.axis_index("s") * NC + lax.axis_index("c")
  base = wid * RPW

  pltpu.sync_copy(tok_hbm.at[pl.ds(base, RPW)], idx_v)

  def do_row(r, _):
    c0 = pltpu.async_copy(
        table_hbm.at[idx_v.at[r, pl.ds(0, CH0)]], buf_v.at[pl.ds(0, CH0)], sem)
    c1 = pltpu.async_copy(
        table_hbm.at[idx_v.at[r, pl.ds(CH0, CH1)]], buf_v.at[pl.ds(CH0, CH1)],
        sem)
    c0.wait()
    c1.wait()

    def acc_body(t, accs):
      return tuple(accs[c] + buf_v[t, pl.ds(c * L, L)] for c in range(NV))

    zeros = tuple(jnp.zeros((L,), jnp.float32) for _ in range(NV))
    accs = lax.fori_loop(0, T, acc_body, zeros)
    for c in range(NV):
      out_v[r, pl.ds(c * L, L)] = accs[c] * SCALE
    return ()

  lax.fori_loop(0, RPW, do_row, ())

  pltpu.sync_copy(out_v, out_hbm.at[pl.ds(base, RPW)])


@functools.partial(jax.jit, static_argnames=())
def _encoder(token_ids, table):
  mesh = plsc.VectorSubcoreMesh(
      core_axis_name="c", subcore_axis_name="s", num_cores=NC,
      num_subcores=NS)

  relayout = pl.kernel(
      _relayout_body,
      out_type=jax.ShapeDtypeStruct((VOCAB, DP), jnp.float32),
      mesh=mesh,
      scratch_types=[
          pltpu.VMEM((D // 8, 8, BLK), jnp.float32),
          pltpu.VMEM((BLK // 8, 8, DP), jnp.float32),
          pltpu.VMEM((TAIL, DP), jnp.float32),
          pltpu.SemaphoreType.DMA,
          pltpu.SemaphoreType.DMA,
      ],
      compiler_params=pltpu.CompilerParams(needs_layout_passes=False),
  )
  tail_pad = jnp.pad(table[VOCAB - TAIL:, :], ((0, 0), (0, DP - D)))
  table_rm = relayout(table.T, tail_pad)

  gather = pl.kernel(
      _gather_body,
      out_type=jax.ShapeDtypeStruct((B, D), jnp.float32),
      mesh=mesh,
      scratch_types=[
          pltpu.VMEM((RPW, T), jnp.int32),
          pltpu.VMEM((T, DP), jnp.float32),
          pltpu.VMEM((RPW, D), jnp.float32),
          pltpu.SemaphoreType.DMA,
      ],
  )
  return gather(token_ids, table_rm)


def kernel(token_ids, table):
  return _encoder(token_ids.astype(jnp.int32), table)


# speedup vs baseline: 1.2783x; 1.2783x over previous
import functools

import jax
import jax.numpy as jnp
from jax import lax
from jax.experimental import pallas as pl
from jax.experimental.pallas import tpu as pltpu
from jax.experimental.pallas import tpu_sc as plsc

VOCAB = 1_000_000
D = 64
DP = 128
B = 4096
T = 200

NC = 2
NS = 16
NW = NC * NS
RPW = B // NW

BLK = 128
NFULL = VOCAB // BLK
TAIL = VOCAB - NFULL * BLK
NG = D // 8
NJ = BLK // 8

CH0, CH1 = 128, 72

L = 16
NV = D // L
SCALE = 1.0 / T


def _relayout_body(src_hbm, tail_hbm, out_hbm, stage_v, outst_v, tail_v,
                   sem_in, sem_out):
  wid = lax.axis_index("s") * NC + lax.axis_index("c")
  lanes = lax.iota(jnp.int32, L)
  idx_g = [(q * L + lanes) // 8 for q in range(NV)]
  idx_c = [(q * L + lanes) % 8 for q in range(NV)]

  def issue_in(b, sl):
    i0 = b * BLK
    for g in range(NG):
      pltpu.async_copy(src_hbm.at[pl.ds(8 * g, 8), pl.ds(i0, BLK)],
                       stage_v.at[sl, g], sem_in)

  def drain_in(sl):
    for g in range(NG):
      pltpu.make_async_copy(src_hbm.at[pl.ds(0, 8), pl.ds(0, BLK)],
                            stage_v.at[sl, g], sem_in).wait()

  def issue_out(b, sl):
    i0 = b * BLK
    for j in range(NJ):
      pltpu.async_copy(outst_v.at[sl, pl.ds(8 * j, 8)],
                       out_hbm.at[pl.ds(i0 + 8 * j, 8)], sem_out)

  def drain_out(sl):
    for j in range(NJ):
      pltpu.make_async_copy(outst_v.at[sl, pl.ds(8 * j, 8)],
                            out_hbm.at[pl.ds(8 * j, 8)], sem_out).wait()

  @pl.when(wid == 0)
  def _tail():
    pltpu.sync_copy(tail_hbm, tail_v)
    pltpu.sync_copy(tail_v, out_hbm.at[pl.ds(NFULL * BLK, TAIL)])

  nper = NFULL // NW + 1

  @pl.when(wid < NFULL)
  def _prime():
    issue_in(wid, 0)

  def do_block(k, _):
    b = wid + k * NW
    sl = lax.rem(k, 2)

    @pl.when(b < NFULL)
    def _full():
      drain_in(sl)

      @pl.when(b + NW < NFULL)
      def _prefetch():
        issue_in(b + NW, 1 - sl)

      @pl.when(k >= 2)
      def _reuse():
        drain_out(sl)

      def grp(gi, _):
        ibase = jnp.zeros((L,), jnp.int32) + gi * 8
        for r in range(8):
          i = gi * 8 + r
          idx_i = ibase + r
          for q in range(NV):
            g = plsc.load_gather(stage_v.at[sl], [idx_g[q], idx_c[q], idx_i])
            outst_v[sl, i, pl.ds(q * L, L)] = g
        return ()

      lax.fori_loop(0, NJ, grp, ())
      issue_out(b, sl)

    return ()

  lax.fori_loop(0, nper, do_block, ())

  nblk = (NFULL - wid + NW - 1) // NW

  @pl.when(nblk >= 1)
  def _d1():
    drain_out(lax.rem(nblk - 1, 2))

  @pl.when(nblk >= 2)
  def _d2():
    drain_out(lax.rem(nblk - 2, 2))


def _gather_body(tok_hbm, table_hbm, out_hbm, idx_v, buf_v, out_v, sem):
  wid = lax.axis_index("s") * NC + lax.axis_index("c")
  base = wid * RPW

  pltpu.sync_copy(tok_hbm.at[pl.ds(base, RPW)], idx_v)

  def issue(r, sl):
    pltpu.async_copy(
        table_hbm.at[idx_v.at[r, pl.ds(0, CH0)]],
        buf_v.at[sl, pl.ds(0, CH0)], sem)
    pltpu.async_copy(
        table_hbm.at[idx_v.at[r, pl.ds(CH0, CH1)]],
        buf_v.at[sl, pl.ds(CH0, CH1)], sem)

  def drain(r, sl):
    pltpu.make_async_copy(
        table_hbm.at[idx_v.at[r, pl.ds(0, CH0)]],
        buf_v.at[sl, pl.ds(0, CH0)], sem).wait()
    pltpu.make_async_copy(
        table_hbm.at[idx_v.at[r, pl.ds(CH0, CH1)]],
        buf_v.at[sl, pl.ds(CH0, CH1)], sem).wait()

  issue(0, 0)

  def do_row(r, _):
    sl = lax.rem(r, 2)
    drain(r, sl)

    @pl.when(r + 1 < RPW)
    def _prefetch():
      issue(r + 1, 1 - sl)

    def acc_grp(tg, accs):
      upd = accs
      for dt in range(8):
        t = tg * 8 + dt
        upd = tuple(upd[c] + buf_v[sl, t, pl.ds(c * L, L)] for c in range(NV))
      return upd

    zeros = tuple(jnp.zeros((L,), jnp.float32) for _ in range(NV))
    accs = lax.fori_loop(0, T // 8, acc_grp, zeros)
    for c in range(NV):
      out_v[r, pl.ds(c * L, L)] = accs[c] * SCALE
    return ()

  lax.fori_loop(0, RPW, do_row, ())

  pltpu.sync_copy(out_v, out_hbm.at[pl.ds(base, RPW)])


@functools.partial(jax.jit, static_argnames=())
def _encoder(token_ids, table):
  mesh = plsc.VectorSubcoreMesh(
      core_axis_name="c", subcore_axis_name="s", num_cores=NC,
      num_subcores=NS)

  relayout = pl.kernel(
      _relayout_body,
      out_type=jax.ShapeDtypeStruct((VOCAB, DP), jnp.float32),
      mesh=mesh,
      scratch_types=[
          pltpu.VMEM((2, NG, 8, BLK), jnp.float32),
          pltpu.VMEM((2, BLK, DP), jnp.float32),
          pltpu.VMEM((TAIL, DP), jnp.float32),
          pltpu.SemaphoreType.DMA,
          pltpu.SemaphoreType.DMA,
      ],
      compiler_params=pltpu.CompilerParams(needs_layout_passes=False),
  )
  tail_pad = jnp.pad(table[VOCAB - TAIL:, :], ((0, 0), (0, DP - D)))
  table_rm = relayout(table.T, tail_pad)

  gather = pl.kernel(
      _gather_body,
      out_type=jax.ShapeDtypeStruct((B, D), jnp.float32),
      mesh=mesh,
      scratch_types=[
          pltpu.VMEM((RPW, T), jnp.int32),
          pltpu.VMEM((2, T, DP), jnp.float32),
          pltpu.VMEM((RPW, D), jnp.float32),
          pltpu.SemaphoreType.DMA,
      ],
  )
  return gather(token_ids, table_rm)


def kernel(token_ids, table):
  return _encoder(token_ids.astype(jnp.int32), table)


# speedup vs baseline: 1.4997x; 1.1732x over previous
import functools

import jax
import jax.numpy as jnp
from jax import lax
from jax.experimental import pallas as pl
from jax.experimental.pallas import tpu as pltpu
from jax.experimental.pallas import tpu_sc as plsc

VOCAB = 1_000_000
D = 64
DP = 128
B = 4096
T = 200

NC = 2
NS = 16
NW = NC * NS
RPW = B // NW

W = 512

CH0, CH1 = 128, 72

L = 16
NV = D // L
SCALE = 1.0 / T


def _xpose_body(x_ref, o_ref):
  xt = x_ref[...].T
  o_ref[:, 0:D] = xt
  o_ref[:, D:DP] = xt


def _gather_body(tok_hbm, table_hbm, out_hbm, idx_v, buf_v, out_v, sem):
  wid = lax.axis_index("s") * NC + lax.axis_index("c")
  base = wid * RPW

  pltpu.sync_copy(tok_hbm.at[pl.ds(base, RPW)], idx_v)

  def issue(r, sl):
    pltpu.async_copy(
        table_hbm.at[idx_v.at[r, pl.ds(0, CH0)]],
        buf_v.at[sl, pl.ds(0, CH0)], sem)
    pltpu.async_copy(
        table_hbm.at[idx_v.at[r, pl.ds(CH0, CH1)]],
        buf_v.at[sl, pl.ds(CH0, CH1)], sem)

  def drain(r, sl):
    pltpu.make_async_copy(
        table_hbm.at[idx_v.at[r, pl.ds(0, CH0)]],
        buf_v.at[sl, pl.ds(0, CH0)], sem).wait()
    pltpu.make_async_copy(
        table_hbm.at[idx_v.at[r, pl.ds(CH0, CH1)]],
        buf_v.at[sl, pl.ds(CH0, CH1)], sem).wait()

  issue(0, 0)

  def do_row(r, _):
    sl = lax.rem(r, 2)
    drain(r, sl)

    @pl.when(r + 1 < RPW)
    def _prefetch():
      issue(r + 1, 1 - sl)

    def acc_grp(tg, accs):
      upd = accs
      for dt in range(8):
        t = tg * 8 + dt
        upd = tuple(upd[c] + buf_v[sl, t, pl.ds(c * L, L)] for c in range(NV))
      return upd

    zeros = tuple(jnp.zeros((L,), jnp.float32) for _ in range(NV))
    accs = lax.fori_loop(0, T // 8, acc_grp, zeros)
    for c in range(NV):
      out_v[r, pl.ds(c * L, L)] = accs[c] * SCALE
    return ()

  lax.fori_loop(0, RPW, do_row, ())

  pltpu.sync_copy(out_v, out_hbm.at[pl.ds(base, RPW)])


@functools.partial(jax.jit, static_argnames=())
def _encoder(token_ids, table):
  nblk = (VOCAB + W - 1) // W
  xpose = pl.pallas_call(
      _xpose_body,
      grid=(nblk,),
      in_specs=[pl.BlockSpec((D, W), lambda i: (0, i))],
      out_specs=pl.BlockSpec((W, DP), lambda i: (i, 0)),
      out_shape=jax.ShapeDtypeStruct((VOCAB, DP), jnp.float32),
      compiler_params=pltpu.CompilerParams(
          dimension_semantics=("arbitrary",)),
  )
  table_rm = xpose(table.T)

  mesh = plsc.VectorSubcoreMesh(
      core_axis_name="c", subcore_axis_name="s", num_cores=NC,
      num_subcores=NS)
  gather = pl.kernel(
      _gather_body,
      out_type=jax.ShapeDtypeStruct((B, D), jnp.float32),
      mesh=mesh,
      scratch_types=[
          pltpu.VMEM((RPW, T), jnp.int32),
          pltpu.VMEM((2, T, DP), jnp.float32),
          pltpu.VMEM((RPW, D), jnp.float32),
          pltpu.SemaphoreType.DMA,
      ],
  )
  return gather(token_ids, table_rm)


def kernel(token_ids, table):
  return _encoder(token_ids.astype(jnp.int32), table)
